# baseline (speedup 1.0000x reference)
import functools

import jax
import jax.numpy as jnp
from jax import lax
from jax.experimental import pallas as pl
from jax.experimental.pallas import tpu as pltpu
from jax.experimental.pallas import tpu_sc as plsc

_NUM_NODES = 1000000
_CHANNELS = 64
_BATCH = 16384
_SLAB = 128
_L = 16


def _make_gather():
    info = plsc.get_sparse_core_info()
    nc, ns = info.num_cores, info.num_subcores
    nw = nc * ns
    b_per_w = _BATCH // nw
    nbuf = 8
    grp = 16

    mesh = plsc.VectorSubcoreMesh(core_axis_name="c", subcore_axis_name="s")

    @functools.partial(
        pl.kernel,
        mesh=mesh,
        out_type=jax.ShapeDtypeStruct((_BATCH, _CHANNELS), jnp.float32),
        scratch_types=[
            pltpu.VMEM((b_per_w,), jnp.int32),
            pltpu.VMEM((nbuf, _CHANNELS, _SLAB), jnp.float32),
            pltpu.VMEM((2, grp, _CHANNELS), jnp.float32),
            pltpu.SemaphoreType.DMA((nbuf,)),
            pltpu.SemaphoreType.DMA((2,)),
        ],
    )
    def gather_kernel(idx_hbm, tab_hbm, out_hbm, idx_v, slab_v, stage_v,
                      gsem, wsem):
        wid = lax.axis_index("s") * nc + lax.axis_index("c")
        base = wid * b_per_w
        pltpu.sync_copy(idx_hbm.at[pl.ds(base, b_per_w)], idx_v)

        lane_iota = lax.iota(jnp.int32, _L)

        def issue(n, buf):
            lane0 = pl.multiple_of((n >> 7) << 7, _SLAB)
            for t in range(_CHANNELS // 8):
                pltpu.async_copy(
                    tab_hbm.at[pl.ds(t * 8, 8), pl.ds(lane0, _SLAB)],
                    slab_v.at[buf, pl.ds(t * 8, 8), :],
                    gsem.at[buf],
                )

        def half_body(h0, sbuf, nvec, nvec_next, prime):
            if prime:
                for j in range(nbuf):
                    issue(nvec[j], j)
            for j in range(grp):
                buf = j % nbuf
                for t in range(_CHANNELS // 8):
                    pltpu.make_async_copy(
                        tab_hbm.at[pl.ds(t * 8, 8), pl.ds(0, _SLAB)],
                        slab_v.at[buf, pl.ds(t * 8, 8), :],
                        gsem.at[buf],
                    ).wait()
                n = nvec[j]
                li = jnp.full((_L,), n & (_L - 1), dtype=jnp.int32)
                ci = ((n & (_SLAB - 1)) >> 4) << 4
                for g in range(_CHANNELS // _L):
                    acc = jnp.zeros((_L,), dtype=jnp.float32)
                    for k in range(_L):
                        v = slab_v[buf, g * _L + k, pl.ds(ci, _L)]
                        p = v.at[li].get(mode="promise_in_bounds",
                                         unique_indices=False)
                        acc = jnp.where(lane_iota == k, p, acc)
                    stage_v[sbuf, j, pl.ds(g * _L, _L)] = acc
                nxt = j + nbuf
                if nxt < grp:
                    issue(nvec[nxt], buf)
                else:

                    @pl.when(h0 + nxt < b_per_w)
                    def _():
                        issue(nvec_next[nxt - grp], buf)

            pltpu.make_async_copy(
                stage_v.at[sbuf],
                out_hbm.at[pl.ds(base + h0, grp), :],
                wsem.at[sbuf],
            ).start()

        def pair_body(p, _):
            h0 = p * (2 * grp)
            nvec0 = idx_v[pl.ds(h0, grp)]
            nvec1 = idx_v[pl.ds(h0 + grp, grp)]
            nvec2 = idx_v[pl.ds((h0 + 2 * grp) % b_per_w, grp)]

            @pl.when(p > 0)
            def _():
                pltpu.make_async_copy(
                    stage_v.at[0],
                    out_hbm.at[pl.ds(base, grp), :],
                    wsem.at[0],
                ).wait()

            half_body(h0, 0, nvec0, nvec1, prime=False)

            @pl.when(p > 0)
            def _():
                pltpu.make_async_copy(
                    stage_v.at[1],
                    out_hbm.at[pl.ds(base, grp), :],
                    wsem.at[1],
                ).wait()

            half_body(h0 + grp, 1, nvec1, nvec2, prime=False)
            return ()

        nvec_first = idx_v[pl.ds(0, grp)]
        for j in range(nbuf):
            issue(nvec_first[j], j)
        lax.fori_loop(0, b_per_w // (2 * grp), pair_body, (), unroll=False)
        for s in range(2):
            pltpu.make_async_copy(
                stage_v.at[s],
                out_hbm.at[pl.ds(base, grp), :],
                wsem.at[s],
            ).wait()

    return gather_kernel


_gather = _make_gather()


def kernel(idx, emb_weight):
    return _gather(idx.astype(jnp.int32), emb_weight.T)


# speedup vs baseline: 1.0102x; 1.0102x over previous
import functools

import jax
import jax.numpy as jnp
from jax import lax
from jax.experimental import pallas as pl
from jax.experimental.pallas import tpu as pltpu
from jax.experimental.pallas import tpu_sc as plsc

_NUM_NODES = 1000000
_CHANNELS = 64
_BATCH = 16384
_SLAB = 128
_L = 16


def _make_gather():
    info = plsc.get_sparse_core_info()
    nc, ns = info.num_cores, info.num_subcores
    nw = nc * ns
    b_per_w = _BATCH // nw
    nbuf = 8
    grp = 16

    mesh = plsc.VectorSubcoreMesh(core_axis_name="c", subcore_axis_name="s")

    @functools.partial(
        pl.kernel,
        mesh=mesh,
        out_type=jax.ShapeDtypeStruct((_BATCH, _CHANNELS), jnp.float32),
        scratch_types=[
            pltpu.VMEM((b_per_w,), jnp.int32),
            pltpu.VMEM((nbuf, _CHANNELS, _SLAB), jnp.float32),
            pltpu.VMEM((2, grp, _CHANNELS), jnp.float32),
            pltpu.SemaphoreType.DMA((nbuf,)),
            pltpu.SemaphoreType.DMA((2,)),
        ],
    )
    def gather_kernel(idx_hbm, tab_hbm, out_hbm, idx_v, slab_v, stage_v,
                      gsem, wsem):
        wid = lax.axis_index("s") * nc + lax.axis_index("c")
        base = wid * b_per_w
        pltpu.sync_copy(idx_hbm.at[pl.ds(base, b_per_w)], idx_v)

        lane_iota = lax.iota(jnp.int32, _L)

        def issue(n, buf):
            lane0 = pl.multiple_of((n >> 7) << 7, _SLAB)
            pltpu.async_copy(
                tab_hbm.at[:, pl.ds(lane0, _SLAB)],
                slab_v.at[buf],
                gsem.at[buf],
            )

        def half_body(h0, sbuf, nvec, nvec_next, prime):
            if prime:
                for j in range(nbuf):
                    issue(nvec[j], j)
            for j in range(grp):
                buf = j % nbuf
                pltpu.make_async_copy(
                    tab_hbm.at[:, pl.ds(0, _SLAB)],
                    slab_v.at[buf],
                    gsem.at[buf],
                ).wait()
                n = nvec[j]
                li = jnp.full((_L,), n & (_L - 1), dtype=jnp.int32)
                ci = ((n & (_SLAB - 1)) >> 4) << 4
                for g in range(_CHANNELS // _L):
                    acc = jnp.zeros((_L,), dtype=jnp.float32)
                    for k in range(_L):
                        v = slab_v[buf, g * _L + k, pl.ds(ci, _L)]
                        p = v.at[li].get(mode="promise_in_bounds",
                                         unique_indices=False)
                        acc = jnp.where(lane_iota == k, p, acc)
                    stage_v[sbuf, j, pl.ds(g * _L, _L)] = acc
                nxt = j + nbuf
                if nxt < grp:
                    issue(nvec[nxt], buf)
                else:

                    @pl.when(h0 + nxt < b_per_w)
                    def _():
                        issue(nvec_next[nxt - grp], buf)

            pltpu.make_async_copy(
                stage_v.at[sbuf],
                out_hbm.at[pl.ds(base + h0, grp), :],
                wsem.at[sbuf],
            ).start()

        def pair_body(p, _):
            h0 = p * (2 * grp)
            nvec0 = idx_v[pl.ds(h0, grp)]
            nvec1 = idx_v[pl.ds(h0 + grp, grp)]
            nvec2 = idx_v[pl.ds((h0 + 2 * grp) % b_per_w, grp)]

            @pl.when(p > 0)
            def _():
                pltpu.make_async_copy(
                    stage_v.at[0],
                    out_hbm.at[pl.ds(base, grp), :],
                    wsem.at[0],
                ).wait()

            half_body(h0, 0, nvec0, nvec1, prime=False)

            @pl.when(p > 0)
            def _():
                pltpu.make_async_copy(
                    stage_v.at[1],
                    out_hbm.at[pl.ds(base, grp), :],
                    wsem.at[1],
                ).wait()

            half_body(h0 + grp, 1, nvec1, nvec2, prime=False)
            return ()

        nvec_first = idx_v[pl.ds(0, grp)]
        for j in range(nbuf):
            issue(nvec_first[j], j)
        lax.fori_loop(0, b_per_w // (2 * grp), pair_body, (), unroll=False)
        for s in range(2):
            pltpu.make_async_copy(
                stage_v.at[s],
                out_hbm.at[pl.ds(base, grp), :],
                wsem.at[s],
            ).wait()

    return gather_kernel


_gather = _make_gather()


def kernel(idx, emb_weight):
    return _gather(idx.astype(jnp.int32), emb_weight.T)
